# Initial kernel scaffold; baseline (speedup 1.0000x reference)
import jax
import jax.numpy as jnp
from jax.experimental import pallas as pl


def kernel(updates, mask):
    raise NotImplementedError("write your pallas kernel here")


# speedup vs baseline: 3.4764x; 3.4764x over previous
import functools

import jax
import jax.numpy as jnp
from jax import lax
from jax.experimental import pallas as pl
from jax.experimental.pallas import tpu as pltpu
from jax.experimental.pallas import tpu_sc as plsc

B, H, W_IN, C = 8, 112, 112, 96
OUT_H, OUT_W = 2 * H, 2 * W_IN
N = B * H * W_IN * C
M = B * OUT_H * OUT_W * C

NTILE = 16
CHUNK = 7 * (1 << 17)
NCHUNK = M // CHUNK
PAD = 128
SLICE = CHUNK // NTILE
TS = N // NTILE
WIN = 6144
NWIN = TS // WIN
VPW = WIN // 16
ZW = SLICE // 4


def _sc_body(idx_hbm, upd_hbm, out_hbm, idx_buf, val_buf, off_buf, zbuf, acc):
    c = lax.axis_index("c")
    s = lax.axis_index("s")

    zero16 = jnp.zeros((16,), jnp.float32)

    def _zb(i, carry):
        zbuf[pl.ds(i * 16, 16)] = zero16
        return carry

    lax.fori_loop(0, ZW // 16, _zb, 0)

    iota = lax.iota(jnp.int32, 16)
    dump = CHUNK + 8 * iota

    def _chunk(k, carry):
        chunk_id = 2 * k + c

        @pl.when(chunk_id < NCHUNK)
        def _():
            base = chunk_id * CHUNK
            for z in range(4):
                pltpu.sync_copy(zbuf, acc.at[pl.ds(s * SLICE + z * ZW, ZW)])
            plsc.subcore_barrier()

            def _win(w, wcarry):
                src = s * TS + w * WIN
                pltpu.sync_copy(idx_hbm.at[pl.ds(src, WIN)], idx_buf)
                pltpu.sync_copy(upd_hbm.at[pl.ds(src, WIN)], val_buf)

                def _vec8(j, vcarry):
                    for t in range(8):
                        v = j * 8 + t
                        u = idx_buf[pl.ds(v * 16, 16)] - base
                        ok = (u >= 0) & (u < CHUNK)
                        off_buf[pl.ds(v * 16, 16)] = jnp.where(ok, u, dump + t)
                    return vcarry

                lax.fori_loop(0, VPW // 8, _vec8, 0)
                pltpu.sync_copy(val_buf, acc.at[off_buf], add=True)
                return wcarry

            lax.fori_loop(0, NWIN, _win, 0)
            plsc.subcore_barrier()
            pltpu.sync_copy(
                acc.at[pl.ds(s * SLICE, SLICE)],
                out_hbm.at[pl.ds(base + s * SLICE, SLICE)],
            )

        return carry

    lax.fori_loop(0, (NCHUNK + 1) // 2, _chunk, 0)


@functools.partial(
    pl.kernel,
    mesh=plsc.VectorSubcoreMesh(core_axis_name="c", subcore_axis_name="s"),
    out_type=jax.ShapeDtypeStruct((M,), jnp.float32),
    scratch_types=[
        pltpu.VMEM((WIN,), jnp.int32),
        pltpu.VMEM((WIN,), jnp.float32),
        pltpu.VMEM((WIN,), jnp.int32),
        pltpu.VMEM((ZW,), jnp.float32),
        pltpu.VMEM_SHARED((CHUNK + PAD,), jnp.float32),
    ],
)
def _scatter_add(idx_hbm, upd_hbm, out_hbm, idx_buf, val_buf, off_buf, zbuf, acc):
    _sc_body(idx_hbm, upd_hbm, out_hbm, idx_buf, val_buf, off_buf, zbuf, acc)


@jax.jit
def kernel(updates, mask):
    idx = mask.reshape(-1).astype(jnp.int32)
    upd = updates.reshape(-1)
    flat = _scatter_add(idx, upd)
    return flat.reshape(-1, OUT_H, OUT_W, C)


# speedup vs baseline: 9.6772x; 2.7837x over previous
import functools

import jax
import jax.numpy as jnp
from jax import lax
from jax.experimental import pallas as pl
from jax.experimental.pallas import tpu as pltpu
from jax.experimental.pallas import tpu_sc as plsc

B, H, W_IN, C = 8, 112, 112, 96
OUT_H, OUT_W = 2 * H, 2 * W_IN
N = B * H * W_IN * C
M = B * OUT_H * OUT_W * C

NTILE = 16
CHUNK = 7 * (1 << 18)
NCHUNK = M // CHUNK
PAD = 128
SLICE = CHUNK // NTILE
TS = N // NTILE
WIN = 2688
NWIN = TS // WIN
VPW = WIN // 16
UNROLL = 8


def _sc_body(idx_hbm, upd_hbm, zeros_hbm, out_hbm,
             idx0, idx1, val0, val1, off0, off1, acc, insem0, insem1):
    c = lax.axis_index("c")
    s = lax.axis_index("s")

    iota = lax.iota(jnp.int32, 16)
    dump_u = plsc.bitcast(CHUNK + 8 * iota, jnp.uint32)
    chunk_u = jnp.full((16,), CHUNK, jnp.uint32)

    bufs = ((idx0, val0, off0, insem0), (idx1, val1, off1, insem1))

    def _issue(w, which):
        ib, vb, _, sem = bufs[which]
        src = s * TS + w * WIN
        pltpu.async_copy(idx_hbm.at[pl.ds(src, WIN)], ib, sem)
        pltpu.async_copy(upd_hbm.at[pl.ds(src, WIN)], vb, sem)

    def _chunk(k, carry):
        chunk_id = 2 * k + c

        @pl.when(chunk_id < NCHUNK)
        def _():
            base = chunk_id * CHUNK
            base_vec = jnp.full((16,), 0, jnp.int32) + base
            _issue(0, 0)
            _issue(1, 1)
            pltpu.sync_copy(zeros_hbm, acc.at[pl.ds(s * SLICE, SLICE)])
            plsc.subcore_barrier()

            def _window(w, which):
                ib, vb, ob, sem = bufs[which]
                src = s * TS + w * WIN
                pltpu.make_async_copy(idx_hbm.at[pl.ds(src, WIN)], ib, sem).wait()
                pltpu.make_async_copy(upd_hbm.at[pl.ds(src, WIN)], vb, sem).wait()

                def _vecs(j, vcarry):
                    for t in range(UNROLL):
                        v = j * UNROLL + t
                        u = plsc.bitcast(ib[pl.ds(v * 16, 16)] - base_vec,
                                         jnp.uint32)
                        off = jnp.minimum(u, dump_u + t)
                        ob[pl.ds(v * 16, 16)] = plsc.bitcast(off, jnp.int32)
                    return vcarry

                lax.fori_loop(0, VPW // UNROLL, _vecs, 0)
                pltpu.sync_copy(vb, acc.at[ob], add=True)

                @pl.when(w + 2 < NWIN)
                def _():
                    _issue(w + 2, which)

            def _win2(j, wcarry):
                _window(2 * j, 0)
                _window(2 * j + 1, 1)
                return wcarry

            lax.fori_loop(0, NWIN // 2, _win2, 0)
            plsc.subcore_barrier()
            pltpu.sync_copy(
                acc.at[pl.ds(s * SLICE, SLICE)],
                out_hbm.at[pl.ds(base + s * SLICE, SLICE)],
            )

        return carry

    lax.fori_loop(0, (NCHUNK + 1) // 2, _chunk, 0)


@functools.partial(
    pl.kernel,
    mesh=plsc.VectorSubcoreMesh(core_axis_name="c", subcore_axis_name="s"),
    out_type=jax.ShapeDtypeStruct((M,), jnp.float32),
    scratch_types=[
        pltpu.VMEM((WIN,), jnp.int32),
        pltpu.VMEM((WIN,), jnp.int32),
        pltpu.VMEM((WIN,), jnp.float32),
        pltpu.VMEM((WIN,), jnp.float32),
        pltpu.VMEM((WIN,), jnp.int32),
        pltpu.VMEM((WIN,), jnp.int32),
        pltpu.VMEM_SHARED((CHUNK + PAD,), jnp.float32),
        pltpu.SemaphoreType.DMA,
        pltpu.SemaphoreType.DMA,
    ],
)
def _scatter_add(idx_hbm, upd_hbm, zeros_hbm, out_hbm,
                 idx0, idx1, val0, val1, off0, off1, acc, insem0, insem1):
    _sc_body(idx_hbm, upd_hbm, zeros_hbm, out_hbm,
             idx0, idx1, val0, val1, off0, off1, acc, insem0, insem1)


@jax.jit
def kernel(updates, mask):
    idx = mask.reshape(-1).astype(jnp.int32)
    upd = updates.reshape(-1)
    zeros = jnp.zeros((SLICE,), jnp.float32)
    flat = _scatter_add(idx, upd, zeros)
    return flat.reshape(-1, OUT_H, OUT_W, C)


# speedup vs baseline: 10.6517x; 1.1007x over previous
import functools

import jax
import jax.numpy as jnp
from jax import lax
from jax.experimental import pallas as pl
from jax.experimental.pallas import tpu as pltpu
from jax.experimental.pallas import tpu_sc as plsc

B, H, W_IN, C = 8, 112, 112, 96
OUT_H, OUT_W = 2 * H, 2 * W_IN
N = B * H * W_IN * C
M = B * OUT_H * OUT_W * C

NTILE = 16
CHUNK = 7 * (1 << 18)
NCHUNK = M // CHUNK
PAD = 128
SLICE = CHUNK // NTILE
TS = N // NTILE
WIN = 2048
NWIN = TS // WIN
VPW = WIN // 16
UNROLL = 16


def _sc_body(idx_hbm, upd_hbm, zeros_hbm, out_hbm,
             ib0, ib1, ib2, vb0, vb1, vb2,
             acc, in0, in1, in2, sc0, sc1, sc2):
    c = lax.axis_index("c")
    s = lax.axis_index("s")

    iota = lax.iota(jnp.int32, 16)
    dump_u = plsc.bitcast(CHUNK + 8 * iota, jnp.uint32)

    sets = ((ib0, vb0, in0, sc0), (ib1, vb1, in1, sc1), (ib2, vb2, in2, sc2))

    def _issue_in(w, t):
        ib, vb, insem, _ = sets[t]
        src = s * TS + w * WIN
        pltpu.async_copy(idx_hbm.at[pl.ds(src, WIN)], ib, insem)
        pltpu.async_copy(upd_hbm.at[pl.ds(src, WIN)], vb, insem)

    def _wait_in(w, t):
        ib, vb, insem, _ = sets[t]
        src = s * TS + w * WIN
        pltpu.make_async_copy(idx_hbm.at[pl.ds(src, WIN)], ib, insem).wait()
        pltpu.make_async_copy(upd_hbm.at[pl.ds(src, WIN)], vb, insem).wait()

    def _drain_scat(t):
        ib, vb, _, scsem = sets[t]
        pltpu.make_async_copy(vb, acc.at[ib], scsem).wait()

    def _chunk(k, carry):
        chunk_id = 2 * k + c

        @pl.when(chunk_id < NCHUNK)
        def _():
            base = chunk_id * CHUNK
            base_vec = jnp.full((16,), 0, jnp.int32) + base
            for t in range(3):
                _issue_in(t, t)
            pltpu.sync_copy(zeros_hbm, acc.at[pl.ds(s * SLICE, SLICE)])
            plsc.subcore_barrier()

            def _step(w, t):
                ib, vb, _, scsem = sets[t]
                _wait_in(w, t)

                def _vecs(j, vcarry):
                    for u_ in range(UNROLL):
                        v = j * UNROLL + u_
                        u = plsc.bitcast(ib[pl.ds(v * 16, 16)] - base_vec,
                                         jnp.uint32)
                        off = jnp.minimum(u, dump_u + u_)
                        ib[pl.ds(v * 16, 16)] = plsc.bitcast(off, jnp.int32)
                    return vcarry

                lax.fori_loop(0, VPW // UNROLL, _vecs, 0)
                pltpu.async_copy(vb, acc.at[ib], scsem, add=True)

                tp = (t + 2) % 3

                @pl.when(w >= 1)
                def _():
                    _drain_scat(tp)

                    @pl.when(w + 2 < NWIN)
                    def _():
                        _issue_in(w + 2, tp)

            def _group(j, wcarry):
                for t in range(3):
                    _step(3 * j + t, t)
                return wcarry

            lax.fori_loop(0, NWIN // 3, _group, 0)
            _drain_scat((NWIN - 1) % 3)
            plsc.subcore_barrier()
            pltpu.sync_copy(
                acc.at[pl.ds(s * SLICE, SLICE)],
                out_hbm.at[pl.ds(base + s * SLICE, SLICE)],
            )

        return carry

    lax.fori_loop(0, (NCHUNK + 1) // 2, _chunk, 0)


@functools.partial(
    pl.kernel,
    mesh=plsc.VectorSubcoreMesh(core_axis_name="c", subcore_axis_name="s"),
    out_type=jax.ShapeDtypeStruct((M,), jnp.float32),
    scratch_types=[
        pltpu.VMEM((WIN,), jnp.int32),
        pltpu.VMEM((WIN,), jnp.int32),
        pltpu.VMEM((WIN,), jnp.int32),
        pltpu.VMEM((WIN,), jnp.float32),
        pltpu.VMEM((WIN,), jnp.float32),
        pltpu.VMEM((WIN,), jnp.float32),
        pltpu.VMEM_SHARED((CHUNK + PAD,), jnp.float32),
        pltpu.SemaphoreType.DMA,
        pltpu.SemaphoreType.DMA,
        pltpu.SemaphoreType.DMA,
        pltpu.SemaphoreType.DMA,
        pltpu.SemaphoreType.DMA,
        pltpu.SemaphoreType.DMA,
    ],
)
def _scatter_add(idx_hbm, upd_hbm, zeros_hbm, out_hbm,
                 ib0, ib1, ib2, vb0, vb1, vb2,
                 acc, in0, in1, in2, sc0, sc1, sc2):
    _sc_body(idx_hbm, upd_hbm, zeros_hbm, out_hbm,
             ib0, ib1, ib2, vb0, vb1, vb2,
             acc, in0, in1, in2, sc0, sc1, sc2)


@jax.jit
def kernel(updates, mask):
    idx = mask.reshape(-1).astype(jnp.int32)
    upd = updates.reshape(-1)
    zeros = jnp.zeros((SLICE,), jnp.float32)
    flat = _scatter_add(idx, upd, zeros)
    return flat.reshape(-1, OUT_H, OUT_W, C)
